# Initial kernel scaffold; baseline (speedup 1.0000x reference)
import jax
import jax.numpy as jnp
from jax.experimental import pallas as pl


def kernel(x_obstacle, x_agent, x_goal, edge_index_oa, edge_index_aa, edge_index_ga, edge_attr_oa, edge_attr_aa, edge_attr_ga, action, ne_W, ne_b, ee_W1, ee_b1, ee_W2, ee_b2, cv_W1, cv_b1, cv_W2, cv_b2, f_W1, f_b1, f_W2, f_b2):
    raise NotImplementedError("write your pallas kernel here")


# speedup vs baseline: 3.0339x; 3.0339x over previous
import functools

import jax
import jax.numpy as jnp
from jax import lax
from jax.experimental import pallas as pl
from jax.experimental.pallas import tpu as pltpu, tpu_sc as plsc

N = 10000
E = 320000
H = 128
DX = 4
DE = 4
DA = 2

W = 32
R = 320
PN = W * R
BT = 2048
PE = 321536
GW = 128
B = 128
BN = 2000

def _mesh():
    return plsc.VectorSubcoreMesh(core_axis_name="c", subcore_axis_name="s")


def _dot(a, b):
    return jnp.dot(a, b, preferred_element_type=jnp.float32)


def _init_embed(xo4, xa4, xg4, neW, neb):
    def body(o4, a4, g4, w, b, oo, oa, og):
        oo[...] = _dot(o4[...], w[0]) + b[0]
        oa[...] = _dot(a4[...], w[1]) + b[1]
        og[...] = _dot(g4[...], w[2]) + b[2]

    sp = pl.BlockSpec((BN, DX), lambda i: (i, 0))
    wsp = pl.BlockSpec((3, DX, H), lambda i: (0, 0, 0))
    bsp = pl.BlockSpec((3, H), lambda i: (0, 0))
    osp = pl.BlockSpec((BN, H), lambda i: (i, 0))
    out = jax.ShapeDtypeStruct((N, H), jnp.float32)
    return pl.pallas_call(
        body, grid=(N // BN,),
        in_specs=[sp, sp, sp, wsp, bsp],
        out_specs=[osp, osp, osp],
        out_shape=[out, out, out],
    )(xo4, xa4, xg4, neW, neb)


def _prep_tables(xo, xa, xg, wstk):
    def body(o, a, g, w, out):
        srcs = [a, o, a, a, a, g]
        for k in range(6):
            out[k] = _dot(srcs[k][...], w[k])

    sp = pl.BlockSpec((BN, H), lambda i: (i, 0))
    return pl.pallas_call(
        body, grid=(N // BN,),
        in_specs=[sp, sp, sp, pl.BlockSpec((6, H, H), lambda i: (0, 0, 0))],
        out_specs=pl.BlockSpec((6, BN, H), lambda i: (0, i, 0)),
        out_shape=jax.ShapeDtypeStruct((6, N, H), jnp.float32),
    )(xo, xa, xg, wstk)


def _msg(GG, attrp, eeW1, eeb1, M2, bc, W2, b2, r):
    nb = PE // BT

    def body(gu, gv, at, w1, b1, m2, bcr, w2, b2r, out):
        h = jnp.maximum(_dot(at[...], w1[...]) + b1[...], 0.0)
        z = gu[...] + gv[...] + _dot(h, m2[...]) + bcr[...]
        out[...] = _dot(jnp.maximum(z, 0.0), w2[...]) + b2r[...]

    full = lambda *s: pl.BlockSpec(s, lambda i: tuple(0 for _ in s))
    return pl.pallas_call(
        body, grid=(nb,),
        in_specs=[
            pl.BlockSpec((BT, H), lambda i, r=r: (2 * r * nb + i, 0)),
            pl.BlockSpec((BT, H), lambda i, r=r: ((2 * r + 1) * nb + i, 0)),
            pl.BlockSpec((BT, DE), lambda i: (i, 0)),
            full(DE, H), full(1, H), full(H, H), full(1, H), full(H, H),
            full(1, H),
        ],
        out_specs=pl.BlockSpec((BT, H), lambda i: (i, 0)),
        out_shape=jax.ShapeDtypeStruct((PE, H), jnp.float32),
    )(GG, GG, attrp, eeW1, eeb1, M2, bc, W2, b2)


def _combine(xa, a0, a1, a2):
    def body(x, b0, b1, b2, out):
        f0 = jnp.where(b0[...] > -1e37, b0[...], 0.0)
        f1 = jnp.where(b1[...] > -1e37, b1[...], 0.0)
        f2 = jnp.where(b2[...] > -1e37, b2[...], 0.0)
        out[...] = x[...] + jnp.maximum(jnp.maximum(f0, f1), f2)

    sp = pl.BlockSpec((BN, H), lambda i: (i, 0))
    return pl.pallas_call(
        body, grid=(N // BN,),
        in_specs=[sp, sp, sp, sp],
        out_specs=sp,
        out_shape=jax.ShapeDtypeStruct((N, H), jnp.float32),
    )(xa, a0, a1, a2)


def _field(xa, action, fwa, fwact, fb1, fw2, fb2):
    def body(x, ac, wa, wc, b1, w2, b2, out):
        hh = jnp.maximum(_dot(x[...], wa[...]) + _dot(ac[...], wc[...]) + b1[...], 0.0)
        out[...] = _dot(hh, w2[...]) + b2[...]

    full = lambda *s: pl.BlockSpec(s, lambda i: tuple(0 for _ in s))
    return pl.pallas_call(
        body, grid=(N // BN,),
        in_specs=[
            pl.BlockSpec((BN, H), lambda i: (i, 0)),
            pl.BlockSpec((BN, DA), lambda i: (i, 0)),
            full(H, H), full(DA, H), full(1, H), full(H, 1), full(1, 1),
        ],
        out_specs=pl.BlockSpec((BN, 1), lambda i: (i, 0)),
        out_shape=jax.ShapeDtypeStruct((N, 1), jnp.float32),
    )(xa, action, fwa, fwact, fb1, fw2, fb2)


def _gather(tbl, gidx):
    total = gidx.shape[1]

    @functools.partial(
        pl.kernel,
        out_type=jax.ShapeDtypeStruct((total, H), jnp.float32),
        mesh=_mesh(),
    )
    def k(tbl_hbm, idx_hbm, out_hbm):
        def body(i_vmem, o_vmem):
            pltpu.sync_copy(tbl_hbm.at[i_vmem.at[0]], o_vmem)

        pltpu.emit_pipeline(
            body,
            grid=(total // GW,),
            in_specs=[pl.BlockSpec((1, GW), index_map=lambda i: (0, i))],
            out_specs=[pl.BlockSpec((GW, H), index_map=lambda i: (i, 0))],
            core_axis_name=("c", "s"),
            dimension_semantics=(pltpu.PARALLEL,),
        )(idx_hbm, out_hbm)

    return k(tbl, gidx)


def _segmax(m, dst_sorted, bounds):
    @functools.partial(
        pl.kernel,
        out_type=jax.ShapeDtypeStruct((PN, H), jnp.float32),
        mesh=_mesh(),
        scratch_types=[
            pltpu.VMEM((R, H), jnp.float32),
            pltpu.VMEM((B, H), jnp.float32),
            pltpu.VMEM((B,), jnp.int32),
            pltpu.VMEM((48,), jnp.int32),
        ],
    )
    def k(m_hbm, dst_hbm, bounds_hbm, out_hbm, acc, mwin, dstw, bnd):
        wid = lax.axis_index("s") * 2 + lax.axis_index("c")
        pltpu.sync_copy(bounds_hbm, bnd)
        bv = bnd[pl.ds(wid, 16)]
        lo_e = bv[0]
        hi_e = bv[1]
        lo_n = wid * R

        @pl.loop(0, R)
        def _(i):
            @pl.loop(0, H, step=16)
            def _(c):
                acc[i, pl.ds(c, 16)] = jnp.full((16,), -1e38, jnp.float32)

        a0 = (lo_e // 8) * 8
        nwin = (hi_e - a0 + B - 1) // B

        def win_body(w, _):
            s = jnp.minimum(a0 + w * B, PE - B)
            pltpu.sync_copy(dst_hbm.at[pl.ds(s, B)], dstw)
            pltpu.sync_copy(m_hbm.at[pl.ds(s, B), :], mwin)

            def grp_body(g, _):
                dv = dstw[pl.ds(g * 16, 16)] - lo_n
                for i in range(16):
                    d = dv[i]
                    ok = jnp.logical_and(d >= 0, d < R)

                    @pl.when(ok)
                    def _(d=d, i=i):
                        @pl.loop(0, H, step=16)
                        def _(c):
                            cur = acc[d, pl.ds(c, 16)]
                            val = mwin[g * 16 + i, pl.ds(c, 16)]
                            acc[d, pl.ds(c, 16)] = jnp.maximum(cur, val)

                return 0

            lax.fori_loop(0, B // 16, grp_body, 0)
            return 0

        lax.fori_loop(0, nwin, win_body, 0)
        pltpu.sync_copy(acc, out_hbm.at[pl.ds(lo_n, R), :])

    return k(m, dst_sorted, bounds)


def kernel(x_obstacle, x_agent, x_goal, edge_index_oa, edge_index_aa, edge_index_ga,
           edge_attr_oa, edge_attr_aa, edge_attr_ga, action,
           ne_W, ne_b, ee_W1, ee_b1, ee_W2, ee_b2,
           cv_W1, cv_b1, cv_W2, cv_b2, f_W1, f_b1, f_W2, f_b2):
    eidx = [edge_index_oa, edge_index_aa, edge_index_ga]
    eattr = [edge_attr_oa, edge_attr_aa, edge_attr_ga]
    PAD = PE - E

    dsts, bounds, attrp, gidx_parts = [], [], [], []
    for r in range(3):
        src, dst = eidx[r][0], eidx[r][1]
        perm = jnp.argsort(dst)
        d = jnp.concatenate([dst[perm], jnp.full((PAD,), N, jnp.int32)])
        s = jnp.concatenate([src[perm], jnp.zeros((PAD,), jnp.int32)])
        a = jnp.concatenate([eattr[r][perm], jnp.zeros((PAD, DE), jnp.float32)])
        dsts.append(d)
        attrp.append(a)
        b = jnp.searchsorted(d, jnp.arange(W + 1, dtype=jnp.int32) * R).astype(jnp.int32)
        bounds.append(jnp.pad(b, (0, 48 - (W + 1))))
        gidx_parts.append(2 * r * N + jnp.minimum(d, N - 1))
        gidx_parts.append((2 * r + 1) * N + jnp.minimum(s, N - 1))
    gidx = jnp.concatenate(gidx_parts).reshape(1, 6 * PE)

    W1a = cv_W1[:, :, :H]
    W1b = cv_W1[:, :, H:2 * H]
    W1c = cv_W1[:, :, 2 * H:]
    M2 = jnp.einsum('rkh,lrhj->lrkj', ee_W2, W1c)
    bc = cv_b1 + jnp.einsum('rk,lrkj->lrj', ee_b2, W1c)

    x_o, x_a, x_g = _init_embed(x_obstacle, x_agent, x_goal, ne_W, ne_b)

    for l in range(3):
        wstk = jnp.stack([W1a[l, 0], W1b[l, 0], W1a[l, 1], W1b[l, 1],
                          W1a[l, 2], W1b[l, 2]])
        tbl = _prep_tables(x_o, x_a, x_g, wstk).reshape(6 * N, H)
        GG = _gather(tbl, gidx)
        aggs = []
        for r in range(3):
            m = _msg(GG, attrp[r], ee_W1[r], ee_b1[r].reshape(1, H),
                     M2[l, r], bc[l, r].reshape(1, H),
                     cv_W2[l, r], cv_b2[l, r].reshape(1, H), r)
            aggs.append(_segmax(m, dsts[r], bounds[r])[:N])
        x_a = _combine(x_a, aggs[0], aggs[1], aggs[2])

    out = _field(x_a, action, f_W1[:H], f_W1[H:], f_b1.reshape(1, H),
                 f_W2, f_b2.reshape(1, 1))
    return out.reshape(N)
